# baseline (speedup 1.0000x reference)
import functools

import jax
import jax.numpy as jnp
from jax import lax
from jax.experimental import pallas as pl
from jax.experimental.pallas import tpu as pltpu
from jax.experimental.pallas import tpu_sc as plsc

VOCAB = 1_000_001
DIM = 64
BATCH = 4096
HIST = 50
TOTAL = BATCH * HIST

_info = plsc.get_sparse_core_info()
NUM_WORKERS = _info.num_cores * _info.num_subcores
B_PER_W = TOTAL // NUM_WORKERS
CHUNK = 128
N_CHUNKS = B_PER_W // CHUNK


def _body(idx_hbm, table_hbm, out_hbm, idx_v, rows_v, sem):
    wid = lax.axis_index("s") * _info.num_cores + lax.axis_index("c")
    base = wid * B_PER_W
    pltpu.sync_copy(idx_hbm.at[pl.ds(base, B_PER_W)], idx_v)

    @pl.loop(0, N_CHUNKS)
    def _chunk(j):
        off = j * CHUNK
        pltpu.async_copy(
            table_hbm.at[idx_v.at[pl.ds(off, CHUNK)]], rows_v, sem
        ).wait()
        pltpu.sync_copy(rows_v, out_hbm.at[pl.ds(base + off, CHUNK)])


@jax.jit
def _embed(idx_flat, table):
    f = pl.kernel(
        _body,
        out_type=jax.ShapeDtypeStruct((TOTAL, DIM), jnp.float32),
        mesh=plsc.VectorSubcoreMesh(core_axis_name="c", subcore_axis_name="s"),
        scratch_types=[
            pltpu.VMEM((B_PER_W,), jnp.int32),
            pltpu.VMEM((CHUNK, DIM), jnp.float32),
            pltpu.SemaphoreType.DMA,
        ],
        compiler_params=pltpu.CompilerParams(use_tc_tiling_on_sc=False),
    )
    return f(idx_flat, table)


def kernel(x, table):
    out = _embed(x.reshape(TOTAL), table)
    return out.reshape(BATCH, HIST, DIM)


# speedup vs baseline: 1.0452x; 1.0452x over previous
import jax
import jax.numpy as jnp
from jax import lax
from jax.experimental import pallas as pl
from jax.experimental.pallas import tpu as pltpu
from jax.experimental.pallas import tpu_sc as plsc

VOCAB = 1_000_001
DIM = 64
BATCH = 4096
HIST = 50
TOTAL = BATCH * HIST

_info = plsc.get_sparse_core_info()
NUM_WORKERS = _info.num_cores * _info.num_subcores
B_PER_W = TOTAL // NUM_WORKERS
STREAM = 128
CHUNK = 640
N_STREAMS = CHUNK // STREAM
N_CHUNKS = B_PER_W // CHUNK


def _body(idx_hbm, table_hbm, out_hbm, idx_v, rows0, rows1, sg0, sg1):
    rows = (rows0, rows1)
    sg = (sg0, sg1)
    wid = lax.axis_index("s") * _info.num_cores + lax.axis_index("c")
    base = wid * B_PER_W
    pltpu.sync_copy(idx_hbm.at[pl.ds(base, B_PER_W)], idx_v)

    def fire_gather(j, b):
        off = j * CHUNK
        for s in range(N_STREAMS):
            pltpu.async_copy(
                table_hbm.at[idx_v.at[pl.ds(off + s * STREAM, STREAM)]],
                rows[b].at[pl.ds(s * STREAM, STREAM)],
                sg[b],
            )

    def wait_gather(b):
        for s in range(N_STREAMS):
            pltpu.make_async_copy(
                table_hbm.at[pl.ds(0, STREAM)],
                rows[b].at[pl.ds(s * STREAM, STREAM)],
                sg[b],
            ).wait()

    def write_out(j, b):
        pltpu.sync_copy(rows[b], out_hbm.at[pl.ds(base + j * CHUNK, CHUNK)])

    fire_gather(0, 0)

    @pl.loop(0, N_CHUNKS - 2, step=2)
    def _pair(j):
        fire_gather(j + 1, 1)
        wait_gather(0)
        write_out(j, 0)
        fire_gather(j + 2, 0)
        wait_gather(1)
        write_out(j + 1, 1)

    fire_gather(N_CHUNKS - 1, 1)
    wait_gather(0)
    write_out(N_CHUNKS - 2, 0)
    wait_gather(1)
    write_out(N_CHUNKS - 1, 1)


@jax.jit
def _embed(idx_flat, table):
    f = pl.kernel(
        _body,
        out_type=jax.ShapeDtypeStruct((TOTAL, DIM), jnp.float32),
        mesh=plsc.VectorSubcoreMesh(core_axis_name="c", subcore_axis_name="s"),
        scratch_types=[
            pltpu.VMEM((B_PER_W,), jnp.int32),
            pltpu.VMEM((CHUNK, DIM), jnp.float32),
            pltpu.VMEM((CHUNK, DIM), jnp.float32),
            pltpu.SemaphoreType.DMA,
            pltpu.SemaphoreType.DMA,
        ],
        compiler_params=pltpu.CompilerParams(use_tc_tiling_on_sc=False),
    )
    return f(idx_flat, table)


def kernel(x, table):
    out = _embed(x.reshape(TOTAL), table)
    return out.reshape(BATCH, HIST, DIM)
